# baseline (speedup 1.0000x reference)
import functools

import jax
import jax.numpy as jnp
from jax import lax
from jax.experimental import pallas as pl
from jax.experimental.pallas import tpu as pltpu
from jax.experimental.pallas import tpu_sc as plsc

N = 10000
IN = 128
OUT = 128
H = 64
K = 16
NB = 8

NPAD = 10240
RB = 512
NT = NPAD // RB
CW = 512
NCH = NPAD // CW
BIG = 1e30

E = N * K
EPAD = NPAD * K
BPW = EPAD // 32
GCH = 128
NGC = BPW // GCH
NBUF = 4

CB = 400
CT = N // CB
EPS = 1e-5


def _premlp_body(x_ref, w1_ref, b1_ref, g1_ref, bb1_ref, w2_ref, b2_ref,
                 g2_ref, bb2_ref, w3_ref, b3_ref, g3_ref, bb3_ref,
                 wu_ref, wb_ref, be1_ref, h_ref, u_ref, v_ref):
  def layer(z, g, bb):
    m = jnp.mean(z, axis=0, keepdims=True)
    v = jnp.mean(z * z, axis=0, keepdims=True) - m * m
    return jnp.maximum(g * (z - m) * lax.rsqrt(v + EPS) + bb, 0.0)

  h = layer(jnp.dot(x_ref[...], w1_ref[...],
                    preferred_element_type=jnp.float32) + b1_ref[...],
            g1_ref[...], bb1_ref[...])
  h = layer(jnp.dot(h, w2_ref[...],
                    preferred_element_type=jnp.float32) + b2_ref[...],
            g2_ref[...], bb2_ref[...])
  h = layer(jnp.dot(h, w3_ref[...],
                    preferred_element_type=jnp.float32) + b3_ref[...],
            g3_ref[...], bb3_ref[...])
  h_ref[...] = h
  u_ref[...] = jnp.dot(h, wu_ref[...],
                       preferred_element_type=jnp.float32) + be1_ref[...]
  v_ref[...] = jnp.dot(h, wb_ref[...], preferred_element_type=jnp.float32)


def _premlp(x, W1, b1, g1, bb1, W2, b2, g2, bb2, W3, b3, g3, bb3, WU, WB,
            be1):
  vec = lambda a: a.reshape(1, -1)
  return pl.pallas_call(
      _premlp_body,
      out_shape=(jax.ShapeDtypeStruct((N, H), jnp.float32),
                 jax.ShapeDtypeStruct((N, OUT), jnp.float32),
                 jax.ShapeDtypeStruct((N, OUT), jnp.float32)),
  )(x, W1, vec(b1), vec(g1), vec(bb1), W2, vec(b2), vec(g2), vec(bb2),
    W3, vec(b3), vec(g3), vec(bb3), WU, WB, vec(be1))


def _extract4(d, ids, npadf, carry):
  ms, as_ = [], []
  for _ in range(4):
    m = jnp.min(d, axis=1, keepdims=True)
    a = jnp.min(jnp.where(d == m, ids, npadf), axis=1, keepdims=True)
    ms.append(m)
    as_.append(a)
    d = jnp.where(ids == a, BIG, d)

  def ce(p, q):
    lo = p[0] <= q[0]
    return ((jnp.minimum(p[0], q[0]), jnp.where(lo, p[1], q[1])),
            (jnp.maximum(p[0], q[0]), jnp.where(lo, q[1], p[1])))

  c = [carry[0:2], carry[2:4], carry[4:6], carry[6:8]]
  ch = [(ms[i], as_[i]) for i in range(4)]
  l = [ce(c[i], ch[3 - i])[0] for i in range(4)]
  l[0], l[2] = ce(l[0], l[2])
  l[1], l[3] = ce(l[1], l[3])
  l[0], l[1] = ce(l[0], l[1])
  l[2], l[3] = ce(l[2], l[3])
  return (l[0][0], l[0][1], l[1][0], l[1][1],
          l[2][0], l[2][1], l[3][0], l[3][1])


def _knn_body(hr_ref, ht_ref, brow_ref, bcol_ref, lo_ref, hi_ref,
              idx_ref, dbuf_ref):
  t = pl.program_id(0)
  c0 = lo_ref[t] // CW
  c1 = (hi_ref[t] + CW - 1) // CW
  hr = hr_ref[...]
  sqr = jnp.sum(hr * hr, axis=1, keepdims=True)
  br = brow_ref[...]
  lane = lax.broadcasted_iota(jnp.int32, (1, CW), 1).astype(jnp.float32)
  npadf = jnp.float32(NPAD)

  def carry0():
    bv0 = jnp.full((RB, 1), 3e38, jnp.float32)
    bi0 = jnp.zeros((RB, 1), jnp.float32)
    return (bv0, bi0) * 4

  def fill(c, carry):
    sl = pl.ds(c * CW, CW)
    hc = ht_ref[:, sl]
    bc = bcol_ref[:, sl]
    dot = lax.dot_general(hr, hc, (((1,), (0,)), ((), ())),
                          preferred_element_type=jnp.float32)
    sqc = jnp.sum(hc * hc, axis=0, keepdims=True)
    d = jnp.where(br == bc, sqr + sqc - 2.0 * dot, BIG)
    dbuf_ref[:, sl] = d
    ids = lane + lax.convert_element_type(c * CW, jnp.float32)
    return _extract4(d, ids, npadf, carry)

  res = lax.fori_loop(c0, c1, fill, carry0())
  picks = [res[1], res[3], res[5], res[7]]
  for s in range(K // 4 - 1):
    prevs = picks[-4:]
    last = s == K // 4 - 2

    def scan(c, carry, prevs=prevs, last=last):
      sl = pl.ds(c * CW, CW)
      ids = lane + lax.convert_element_type(c * CW, jnp.float32)
      d = dbuf_ref[:, sl]
      hit = ((ids == prevs[0]) | (ids == prevs[1]) |
             (ids == prevs[2]) | (ids == prevs[3]))
      d = jnp.where(hit, BIG, d)
      if not last:
        dbuf_ref[:, sl] = d
      return _extract4(d, ids, npadf, carry)

    res = lax.fori_loop(c0, c1, scan, carry0())
    picks.extend([res[1], res[3], res[5], res[7]])
  idx_ref[...] = jnp.concatenate(picks, axis=1).astype(jnp.int32)


def _knn(h_pad, ht_pad, brow, bcol, tile_lo, tile_hi):
  return pl.pallas_call(
      _knn_body,
      grid=(NT,),
      in_specs=[
          pl.BlockSpec((RB, H), lambda i: (i, 0)),
          pl.BlockSpec((H, NPAD), lambda i: (0, 0)),
          pl.BlockSpec((RB, 1), lambda i: (i, 0)),
          pl.BlockSpec((1, NPAD), lambda i: (0, 0)),
          pl.BlockSpec(memory_space=pltpu.SMEM),
          pl.BlockSpec(memory_space=pltpu.SMEM),
      ],
      out_specs=pl.BlockSpec((RB, K), lambda i: (i, 0)),
      out_shape=jax.ShapeDtypeStruct((NPAD, K), jnp.int32),
      scratch_shapes=[pltpu.VMEM((RB, NPAD), jnp.float32)],
  )(h_pad, ht_pad, brow, bcol, tile_lo, tile_hi)


def _gather_rows(v, idx_flat):
  mesh = plsc.VectorSubcoreMesh(core_axis_name="c", subcore_axis_name="s")

  @functools.partial(
      pl.kernel, mesh=mesh,
      out_type=jax.ShapeDtypeStruct((EPAD, OUT), jnp.float32),
      scratch_types=[pltpu.VMEM((BPW,), jnp.int32)]
                    + [pltpu.VMEM((GCH, OUT), jnp.float32)] * NBUF
                    + [pltpu.SemaphoreType.DMA] * NBUF,
  )
  def gk(v_hbm, idx_hbm, out_hbm, idx_v, r0, r1, r2, r3, s0, s1, s2, s3):
    bufs = (r0, r1, r2, r3)
    sems = (s0, s1, s2, s3)
    wid = lax.axis_index("s") * 2 + lax.axis_index("c")
    base = wid * BPW
    pltpu.sync_copy(idx_hbm.at[pl.ds(base, BPW)], idx_v)
    for b in range(NBUF):
      pltpu.async_copy(v_hbm.at[idx_v.at[pl.ds(b * GCH, GCH)]],
                       bufs[b], sems[b])

    def outer(g, _):
      for b in range(NBUF):
        j = g * NBUF + b
        pltpu.make_async_copy(v_hbm.at[idx_v.at[pl.ds(0, GCH)]],
                              bufs[b], sems[b]).wait()
        pltpu.sync_copy(bufs[b], out_hbm.at[pl.ds(base + j * GCH, GCH)])

        @pl.when(j + NBUF < NGC)
        def _():
          pltpu.async_copy(
              v_hbm.at[idx_v.at[pl.ds((j + NBUF) * GCH, GCH)]],
              bufs[b], sems[b])
      return 0

    lax.fori_loop(0, NGC // NBUF, outer, 0)

  return gk(v, idx_flat)


def _estats_body(vg_ref, u_ref, st_ref):
  u = u_ref[...]
  s = jnp.zeros((1, OUT), jnp.float32)
  s2 = jnp.zeros((1, OUT), jnp.float32)
  for k in range(K):
    p = vg_ref[k] + u
    s = s + jnp.sum(p, axis=0, keepdims=True)
    s2 = s2 + jnp.sum(p * p, axis=0, keepdims=True)
  rows = jnp.concatenate([s, s2, jnp.zeros((6, OUT), jnp.float32)], axis=0)
  st_ref[...] = jnp.where(pl.program_id(0) == 0, rows, st_ref[...] + rows)


def _estats(vg3, u):
  return pl.pallas_call(
      _estats_body,
      grid=(CT,),
      in_specs=[
          pl.BlockSpec((K, CB, OUT), lambda i: (0, i, 0)),
          pl.BlockSpec((CB, OUT), lambda i: (i, 0)),
      ],
      out_specs=pl.BlockSpec((8, OUT), lambda i: (0, 0)),
      out_shape=jax.ShapeDtypeStruct((8, OUT), jnp.float32),
  )(vg3, u)


def _edge_body(vg_ref, u_ref, w2_ref, be2_ref, ge_ref, bbe_ref,
               st1_ref, m_ref, st2_ref):
  ne = float(E)
  mean = st1_ref[0:1, :] / ne
  var = st1_ref[1:2, :] / ne - mean * mean
  a = ge_ref[...] * lax.rsqrt(var + EPS)
  c = bbe_ref[...] - mean * a
  u = u_ref[...]
  macc = jnp.full((CB, OUT), -3e38, jnp.float32)
  for k in range(K):
    p = vg_ref[k] + u
    e = jnp.maximum(a * p + c, 0.0)
    e2 = jnp.dot(e, w2_ref[...], preferred_element_type=jnp.float32)
    macc = jnp.maximum(macc, e2)
  m = macc + be2_ref[...]
  m_ref[...] = m
  s = jnp.sum(m, axis=0, keepdims=True)
  s2 = jnp.sum(m * m, axis=0, keepdims=True)
  rows = jnp.concatenate([s, s2, jnp.zeros((6, OUT), jnp.float32)], axis=0)
  st2_ref[...] = jnp.where(pl.program_id(0) == 0, rows, st2_ref[...] + rows)


def _edge(vg3, u, We2, be2, ge, bbe, st1):
  vec = lambda a: a.reshape(1, -1)
  return pl.pallas_call(
      _edge_body,
      grid=(CT,),
      in_specs=[
          pl.BlockSpec((K, CB, OUT), lambda i: (0, i, 0)),
          pl.BlockSpec((CB, OUT), lambda i: (i, 0)),
          pl.BlockSpec((OUT, OUT), lambda i: (0, 0)),
          pl.BlockSpec((1, OUT), lambda i: (0, 0)),
          pl.BlockSpec((1, OUT), lambda i: (0, 0)),
          pl.BlockSpec((1, OUT), lambda i: (0, 0)),
          pl.BlockSpec((8, OUT), lambda i: (0, 0)),
      ],
      out_specs=(pl.BlockSpec((CB, OUT), lambda i: (i, 0)),
                 pl.BlockSpec((8, OUT), lambda i: (0, 0))),
      out_shape=(jax.ShapeDtypeStruct((N, OUT), jnp.float32),
                 jax.ShapeDtypeStruct((8, OUT), jnp.float32)),
  )(vg3, u, We2, vec(be2), vec(ge), vec(bbe), st1)


def _final_body(m_ref, st_ref, gp_ref, bbp_ref, o_ref):
  nn = float(N)
  mean = st_ref[0:1, :] / nn
  var = st_ref[1:2, :] / nn - mean * mean
  a = gp_ref[...] * lax.rsqrt(var + EPS)
  c = bbp_ref[...] - mean * a
  o_ref[...] = jnp.maximum(a * m_ref[...] + c, 0.0)


def _final(m, st2, gp, bbp):
  vec = lambda a: a.reshape(1, -1)
  return pl.pallas_call(
      _final_body,
      grid=(CT,),
      in_specs=[
          pl.BlockSpec((CB, OUT), lambda i: (i, 0)),
          pl.BlockSpec((8, OUT), lambda i: (0, 0)),
          pl.BlockSpec((1, OUT), lambda i: (0, 0)),
          pl.BlockSpec((1, OUT), lambda i: (0, 0)),
      ],
      out_specs=pl.BlockSpec((CB, OUT), lambda i: (i, 0)),
      out_shape=jax.ShapeDtypeStruct((N, OUT), jnp.float32),
  )(m, st2, vec(gp), vec(bbp))


def kernel(x, batch, W1, b1, g1, bb1, W2, b2, g2, bb2, W3, b3, g3, bb3,
           We1, be1, ge, bbe, We2, be2, gp, bbp):
  WA = We1[:H]
  WB = We1[H:]
  WU = WA - WB

  h, U, V = _premlp(x, W1, b1, g1, bb1, W2, b2, g2, bb2, W3, b3, g3, bb3,
                    WU, WB, be1)

  batch = batch.astype(jnp.int32)
  seg_start = jnp.searchsorted(batch, jnp.arange(NB, dtype=jnp.int32),
                               side="left").astype(jnp.int32)
  seg_end = jnp.searchsorted(batch, jnp.arange(NB, dtype=jnp.int32),
                             side="right").astype(jnp.int32)
  row_lo = seg_start[batch]
  row_hi = seg_end[batch]
  pad = NPAD - N
  row_lo = jnp.concatenate([row_lo, jnp.full((pad,), 2**30, jnp.int32)])
  row_hi = jnp.concatenate([row_hi, jnp.zeros((pad,), jnp.int32)])
  tile_lo = jnp.min(row_lo.reshape(NT, RB), axis=1)
  tile_hi = jnp.max(row_hi.reshape(NT, RB), axis=1)

  bf = batch.astype(jnp.float32)
  bf = jnp.concatenate([bf, jnp.full((pad,), -1.0, jnp.float32)])
  h_pad = jnp.concatenate([h, jnp.zeros((pad, H), jnp.float32)])

  idx = _knn(h_pad, h_pad.T, bf.reshape(NPAD, 1), bf.reshape(1, NPAD),
             tile_lo, tile_hi)

  idx_flat = idx.T.reshape(-1)

  vg = _gather_rows(V, idx_flat)
  vg3 = vg.reshape(K, NPAD, OUT)

  st1 = _estats(vg3, U)
  m, st2 = _edge(vg3, U, We2, be2, ge, bbe, st1)
  return _final(m, st2, gp, bbp)


# speedup vs baseline: 1.0882x; 1.0882x over previous
import functools

import jax
import jax.numpy as jnp
from jax import lax
from jax.experimental import pallas as pl
from jax.experimental.pallas import tpu as pltpu
from jax.experimental.pallas import tpu_sc as plsc

N = 10000
IN = 128
OUT = 128
H = 64
K = 16
NB = 8

NPAD = 10240
RB = 512
NT = NPAD // RB
CW = 512
NCH = NPAD // CW
BIG = 1e30

E = N * K
EPAD = NPAD * K
BPW = EPAD // 32
GCH = 128
NGC = BPW // GCH
NBUF = 4

CB = 400
CT = N // CB
EPS = 1e-5


def _premlp_body(x_ref, w1_ref, b1_ref, g1_ref, bb1_ref, w2_ref, b2_ref,
                 g2_ref, bb2_ref, w3_ref, b3_ref, g3_ref, bb3_ref,
                 wu_ref, wb_ref, be1_ref, h_ref, u_ref, v_ref):
  def layer(z, g, bb):
    m = jnp.mean(z, axis=0, keepdims=True)
    v = jnp.mean(z * z, axis=0, keepdims=True) - m * m
    return jnp.maximum(g * (z - m) * lax.rsqrt(v + EPS) + bb, 0.0)

  h = layer(jnp.dot(x_ref[...], w1_ref[...],
                    preferred_element_type=jnp.float32) + b1_ref[...],
            g1_ref[...], bb1_ref[...])
  h = layer(jnp.dot(h, w2_ref[...],
                    preferred_element_type=jnp.float32) + b2_ref[...],
            g2_ref[...], bb2_ref[...])
  h = layer(jnp.dot(h, w3_ref[...],
                    preferred_element_type=jnp.float32) + b3_ref[...],
            g3_ref[...], bb3_ref[...])
  h_ref[...] = h
  u_ref[...] = jnp.dot(h, wu_ref[...],
                       preferred_element_type=jnp.float32) + be1_ref[...]
  v_ref[...] = jnp.dot(h, wb_ref[...], preferred_element_type=jnp.float32)


def _premlp(x, W1, b1, g1, bb1, W2, b2, g2, bb2, W3, b3, g3, bb3, WU, WB,
            be1):
  vec = lambda a: a.reshape(1, -1)
  return pl.pallas_call(
      _premlp_body,
      out_shape=(jax.ShapeDtypeStruct((N, H), jnp.float32),
                 jax.ShapeDtypeStruct((N, OUT), jnp.float32),
                 jax.ShapeDtypeStruct((N, OUT), jnp.float32)),
  )(x, W1, vec(b1), vec(g1), vec(bb1), W2, vec(b2), vec(g2), vec(bb2),
    W3, vec(b3), vec(g3), vec(bb3), WU, WB, vec(be1))


def _extract2(d, ids, npadf, carry):
  bv1, bi1, bv2, bi2 = carry
  m1 = jnp.min(d, axis=1, keepdims=True)
  a1 = jnp.min(jnp.where(d == m1, ids, npadf), axis=1, keepdims=True)
  d2 = jnp.where(ids == a1, BIG, d)
  m2 = jnp.min(d2, axis=1, keepdims=True)
  a2 = jnp.min(jnp.where(d2 == m2, ids, npadf), axis=1, keepdims=True)
  take1 = m1 < bv1
  n1v = jnp.where(take1, m1, bv1)
  n1i = jnp.where(take1, a1, bi1)
  altv = jnp.where(take1, bv1, m1)
  alti = jnp.where(take1, bi1, a1)
  othv = jnp.where(take1, m2, bv2)
  othi = jnp.where(take1, a2, bi2)
  n2v = jnp.minimum(altv, othv)
  n2i = jnp.where(altv <= othv, alti, othi)
  return n1v, n1i, n2v, n2i


def _knn_body(hr_ref, ht_ref, brow_ref, bcol_ref, lo_ref, hi_ref,
              idx_ref, dbuf_ref):
  t = pl.program_id(0)
  c0 = lo_ref[t] // CW
  c1 = (hi_ref[t] + CW - 1) // CW
  hr = hr_ref[...]
  sqr = jnp.sum(hr * hr, axis=1, keepdims=True)
  br = brow_ref[...]
  lane = lax.broadcasted_iota(jnp.int32, (1, CW), 1).astype(jnp.float32)
  npadf = jnp.float32(NPAD)

  def carry0():
    bv0 = jnp.full((RB, 1), 3e38, jnp.float32)
    bi0 = jnp.zeros((RB, 1), jnp.float32)
    return (bv0, bi0, bv0, bi0)

  def fill(c, carry):
    sl = pl.ds(c * CW, CW)
    hc = ht_ref[:, sl]
    bc = bcol_ref[:, sl]
    dot = lax.dot_general(hr, hc, (((1,), (0,)), ((), ())),
                          preferred_element_type=jnp.float32)
    sqc = jnp.sum(hc * hc, axis=0, keepdims=True)
    d = jnp.where(br == bc, sqr + sqc - 2.0 * dot, BIG)
    dbuf_ref[:, sl] = d
    ids = lane + lax.convert_element_type(c * CW, jnp.float32)
    return _extract2(d, ids, npadf, carry)

  res = lax.fori_loop(c0, c1, fill, carry0())
  picks = [res[1], res[3]]
  for s in range(K // 2 - 1):
    prev1, prev2 = picks[-2], picks[-1]
    last = s == K // 2 - 2

    def scan(c, carry, prev1=prev1, prev2=prev2, last=last):
      sl = pl.ds(c * CW, CW)
      ids = lane + lax.convert_element_type(c * CW, jnp.float32)
      d = dbuf_ref[:, sl]
      d = jnp.where((ids == prev1) | (ids == prev2), BIG, d)
      if not last:
        dbuf_ref[:, sl] = d
      return _extract2(d, ids, npadf, carry)

    res = lax.fori_loop(c0, c1, scan, carry0())
    picks.extend([res[1], res[3]])
  idx_ref[...] = jnp.concatenate(picks, axis=1).astype(jnp.int32)


def _knn(h_pad, ht_pad, brow, bcol, tile_lo, tile_hi):
  return pl.pallas_call(
      _knn_body,
      grid=(NT,),
      in_specs=[
          pl.BlockSpec((RB, H), lambda i: (i, 0)),
          pl.BlockSpec((H, NPAD), lambda i: (0, 0)),
          pl.BlockSpec((RB, 1), lambda i: (i, 0)),
          pl.BlockSpec((1, NPAD), lambda i: (0, 0)),
          pl.BlockSpec(memory_space=pltpu.SMEM),
          pl.BlockSpec(memory_space=pltpu.SMEM),
      ],
      out_specs=pl.BlockSpec((RB, K), lambda i: (i, 0)),
      out_shape=jax.ShapeDtypeStruct((NPAD, K), jnp.int32),
      scratch_shapes=[pltpu.VMEM((RB, NPAD), jnp.float32)],
  )(h_pad, ht_pad, brow, bcol, tile_lo, tile_hi)


def _gather_rows(v, idx_flat):
  mesh = plsc.VectorSubcoreMesh(core_axis_name="c", subcore_axis_name="s")

  @functools.partial(
      pl.kernel, mesh=mesh,
      out_type=jax.ShapeDtypeStruct((EPAD, OUT), jnp.float32),
      scratch_types=[pltpu.VMEM((BPW,), jnp.int32)]
                    + [pltpu.VMEM((GCH, OUT), jnp.float32)] * NBUF
                    + [pltpu.SemaphoreType.DMA] * NBUF,
  )
  def gk(v_hbm, idx_hbm, out_hbm, idx_v, r0, r1, r2, r3, s0, s1, s2, s3):
    bufs = (r0, r1, r2, r3)
    sems = (s0, s1, s2, s3)
    wid = lax.axis_index("s") * 2 + lax.axis_index("c")
    base = wid * BPW
    pltpu.sync_copy(idx_hbm.at[pl.ds(base, BPW)], idx_v)
    for b in range(NBUF):
      pltpu.async_copy(v_hbm.at[idx_v.at[pl.ds(b * GCH, GCH)]],
                       bufs[b], sems[b])

    def outer(g, _):
      for b in range(NBUF):
        j = g * NBUF + b
        pltpu.make_async_copy(v_hbm.at[idx_v.at[pl.ds(0, GCH)]],
                              bufs[b], sems[b]).wait()
        pltpu.sync_copy(bufs[b], out_hbm.at[pl.ds(base + j * GCH, GCH)])

        @pl.when(j + NBUF < NGC)
        def _():
          pltpu.async_copy(
              v_hbm.at[idx_v.at[pl.ds((j + NBUF) * GCH, GCH)]],
              bufs[b], sems[b])
      return 0

    lax.fori_loop(0, NGC // NBUF, outer, 0)

  return gk(v, idx_flat)


def _estats_body(vg_ref, u_ref, st_ref):
  u = u_ref[...]
  s = jnp.zeros((1, OUT), jnp.float32)
  s2 = jnp.zeros((1, OUT), jnp.float32)
  for k in range(K):
    p = vg_ref[k] + u
    s = s + jnp.sum(p, axis=0, keepdims=True)
    s2 = s2 + jnp.sum(p * p, axis=0, keepdims=True)
  rows = jnp.concatenate([s, s2, jnp.zeros((6, OUT), jnp.float32)], axis=0)
  st_ref[...] = jnp.where(pl.program_id(0) == 0, rows, st_ref[...] + rows)


def _estats(vg3, u):
  return pl.pallas_call(
      _estats_body,
      grid=(CT,),
      in_specs=[
          pl.BlockSpec((K, CB, OUT), lambda i: (0, i, 0)),
          pl.BlockSpec((CB, OUT), lambda i: (i, 0)),
      ],
      out_specs=pl.BlockSpec((8, OUT), lambda i: (0, 0)),
      out_shape=jax.ShapeDtypeStruct((8, OUT), jnp.float32),
  )(vg3, u)


def _edge_body(vg_ref, u_ref, w2_ref, be2_ref, ge_ref, bbe_ref,
               st1_ref, m_ref, st2_ref):
  ne = float(E)
  mean = st1_ref[0:1, :] / ne
  var = st1_ref[1:2, :] / ne - mean * mean
  a = ge_ref[...] * lax.rsqrt(var + EPS)
  c = bbe_ref[...] - mean * a
  u = u_ref[...]
  macc = jnp.full((CB, OUT), -3e38, jnp.float32)
  for k in range(K):
    p = vg_ref[k] + u
    e = jnp.maximum(a * p + c, 0.0)
    e2 = jnp.dot(e, w2_ref[...], preferred_element_type=jnp.float32)
    macc = jnp.maximum(macc, e2)
  m = macc + be2_ref[...]
  m_ref[...] = m
  s = jnp.sum(m, axis=0, keepdims=True)
  s2 = jnp.sum(m * m, axis=0, keepdims=True)
  rows = jnp.concatenate([s, s2, jnp.zeros((6, OUT), jnp.float32)], axis=0)
  st2_ref[...] = jnp.where(pl.program_id(0) == 0, rows, st2_ref[...] + rows)


def _edge(vg3, u, We2, be2, ge, bbe, st1):
  vec = lambda a: a.reshape(1, -1)
  return pl.pallas_call(
      _edge_body,
      grid=(CT,),
      in_specs=[
          pl.BlockSpec((K, CB, OUT), lambda i: (0, i, 0)),
          pl.BlockSpec((CB, OUT), lambda i: (i, 0)),
          pl.BlockSpec((OUT, OUT), lambda i: (0, 0)),
          pl.BlockSpec((1, OUT), lambda i: (0, 0)),
          pl.BlockSpec((1, OUT), lambda i: (0, 0)),
          pl.BlockSpec((1, OUT), lambda i: (0, 0)),
          pl.BlockSpec((8, OUT), lambda i: (0, 0)),
      ],
      out_specs=(pl.BlockSpec((CB, OUT), lambda i: (i, 0)),
                 pl.BlockSpec((8, OUT), lambda i: (0, 0))),
      out_shape=(jax.ShapeDtypeStruct((N, OUT), jnp.float32),
                 jax.ShapeDtypeStruct((8, OUT), jnp.float32)),
  )(vg3, u, We2, vec(be2), vec(ge), vec(bbe), st1)


def _final_body(m_ref, st_ref, gp_ref, bbp_ref, o_ref):
  nn = float(N)
  mean = st_ref[0:1, :] / nn
  var = st_ref[1:2, :] / nn - mean * mean
  a = gp_ref[...] * lax.rsqrt(var + EPS)
  c = bbp_ref[...] - mean * a
  o_ref[...] = jnp.maximum(a * m_ref[...] + c, 0.0)


def _final(m, st2, gp, bbp):
  vec = lambda a: a.reshape(1, -1)
  return pl.pallas_call(
      _final_body,
      grid=(CT,),
      in_specs=[
          pl.BlockSpec((CB, OUT), lambda i: (i, 0)),
          pl.BlockSpec((8, OUT), lambda i: (0, 0)),
          pl.BlockSpec((1, OUT), lambda i: (0, 0)),
          pl.BlockSpec((1, OUT), lambda i: (0, 0)),
      ],
      out_specs=pl.BlockSpec((CB, OUT), lambda i: (i, 0)),
      out_shape=jax.ShapeDtypeStruct((N, OUT), jnp.float32),
  )(m, st2, vec(gp), vec(bbp))


def kernel(x, batch, W1, b1, g1, bb1, W2, b2, g2, bb2, W3, b3, g3, bb3,
           We1, be1, ge, bbe, We2, be2, gp, bbp):
  WA = We1[:H]
  WB = We1[H:]
  WU = WA - WB

  h, U, V = _premlp(x, W1, b1, g1, bb1, W2, b2, g2, bb2, W3, b3, g3, bb3,
                    WU, WB, be1)

  batch = batch.astype(jnp.int32)
  seg_start = jnp.searchsorted(batch, jnp.arange(NB, dtype=jnp.int32),
                               side="left").astype(jnp.int32)
  seg_end = jnp.searchsorted(batch, jnp.arange(NB, dtype=jnp.int32),
                             side="right").astype(jnp.int32)
  row_lo = seg_start[batch]
  row_hi = seg_end[batch]
  pad = NPAD - N
  row_lo = jnp.concatenate([row_lo, jnp.full((pad,), 2**30, jnp.int32)])
  row_hi = jnp.concatenate([row_hi, jnp.zeros((pad,), jnp.int32)])
  tile_lo = jnp.min(row_lo.reshape(NT, RB), axis=1)
  tile_hi = jnp.max(row_hi.reshape(NT, RB), axis=1)

  bf = batch.astype(jnp.float32)
  bf = jnp.concatenate([bf, jnp.full((pad,), -1.0, jnp.float32)])
  h_pad = jnp.concatenate([h, jnp.zeros((pad, H), jnp.float32)])

  idx = _knn(h_pad, h_pad.T, bf.reshape(NPAD, 1), bf.reshape(1, NPAD),
             tile_lo, tile_hi)

  idx_flat = idx.T.reshape(-1)

  vg = _gather_rows(V, idx_flat)
  vg3 = vg.reshape(K, NPAD, OUT)

  st1 = _estats(vg3, U)
  m, st2 = _edge(vg3, U, We2, be2, ge, bbe, st1)
  return _final(m, st2, gp, bbp)
